# Initial kernel scaffold; baseline (speedup 1.0000x reference)
import jax
import jax.numpy as jnp
from jax.experimental import pallas as pl


def kernel(weights, existing_bins, nears, fars):
    raise NotImplementedError("write your pallas kernel here")


# speedup vs baseline: 10.7806x; 10.7806x over previous
import functools

import jax
import jax.numpy as jnp
from jax import lax
from jax.experimental import pallas as pl
from jax.experimental.pallas import tpu as pltpu
from jax.experimental.pallas import tpu_sc as plsc

NUM_RAYS = 65536
NS = 128
NB = NS + 1
NOUT = 2 * NB
NVREG = 16
NOUT_PAD = 272
NW = 32
RAYS_PER_W = NUM_RAYS // NW
CB = 16
NCHUNK = RAYS_PER_W // CB

HIST_PAD = 0.01
EPS = 1e-5

_mesh = plsc.VectorSubcoreMesh(
    core_axis_name="c", subcore_axis_name="s", num_cores=2, num_subcores=16
)


@functools.partial(
    pl.kernel,
    out_type=jax.ShapeDtypeStruct((NUM_RAYS, NOUT), jnp.float32),
    mesh=_mesh,
    compiler_params=pltpu.CompilerParams(needs_layout_passes=False),
    scratch_types=[
        pltpu.VMEM((144,), jnp.float32),
        pltpu.VMEM((CB, NS), jnp.float32),
        pltpu.VMEM((CB, NB), jnp.float32),
        pltpu.VMEM((CB,), jnp.float32),
        pltpu.VMEM((CB,), jnp.float32),
        pltpu.VMEM((CB, NS), jnp.float32),
        pltpu.VMEM((CB, NOUT_PAD), jnp.int32),
        pltpu.VMEM((CB, NOUT), jnp.float32),
    ],
)
def _sampler(w_hbm, eb_hbm, ne_hbm, fa_hbm, u_hbm, out_hbm,
             u_v, w_v, eb_v, ne_v, fa_v, cdf_v, a_v, out_v):
    wid = lax.axis_index("s") * 2 + lax.axis_index("c")
    base0 = wid * RAYS_PER_W

    pltpu.sync_copy(u_hbm, u_v)

    iota = lax.broadcasted_iota(jnp.int32, (NVREG,), 0)
    ones_i = jnp.ones((NVREG,), jnp.int32)
    zeros_i = jnp.zeros((NVREG,), jnp.int32)
    seed0 = jnp.where(iota == 0, 1, 0).astype(jnp.int32)

    def chunk_body(ci, _):
        base = base0 + ci * CB
        pltpu.sync_copy(w_hbm.at[pl.ds(base, CB)], w_v)
        pltpu.sync_copy(eb_hbm.at[pl.ds(base, CB)], eb_v)
        pltpu.sync_copy(ne_hbm.at[pl.ds(base, CB)], ne_v)
        pltpu.sync_copy(fa_hbm.at[pl.ds(base, CB)], fa_v)

        def ray_body(r, carry_unused):
            row = jnp.full((NVREG,), r, jnp.int32)

            a_v[r, pl.ds(0, NVREG)] = seed0
            for q in range(1, NOUT_PAD // NVREG):
                a_v[r, pl.ds(q * NVREG, NVREG)] = zeros_i

            wv = [w_v[r, pl.ds(v * NVREG, NVREG)] + HIST_PAD
                  for v in range(NS // NVREG)]
            acc = wv[0]
            for v in range(1, NS // NVREG):
                acc = acc + wv[v]
            w_sum = jnp.sum(acc)
            pad = jnp.maximum(EPS - w_sum, 0.0)
            w_sum2 = w_sum + pad
            padc = pad * (1.0 / NS)
            carry = jnp.float32(0.0)
            for v in range(NS // NVREG):
                pdf = (wv[v] + padc) / w_sum2
                cs = plsc.cumsum(pdf) + carry
                carry = cs[NVREG - 1]
                cdfv = jnp.minimum(cs, 1.0)
                cdf_v[r, pl.ds(v * NVREG, NVREG)] = cdfv
                y = cdfv * jnp.float32(NB) - 0.5
                tr = y.astype(jnp.int32)
                m = tr + jnp.where(tr.astype(jnp.float32) < y, 1, 0)
                pos = iota + (v * NVREG + 1) + m
                plsc.store_scatter(a_v, [row, pos], ones_i)

            near = plsc.load_gather(ne_v, [row])
            far = plsc.load_gather(fa_v, [row])
            icarry = jnp.int32(0)
            for q in range(NOUT_PAD // NVREG):
                a = a_v[r, pl.ds(q * NVREG, NVREG)]
                c = plsc.cumsum(a) + icarry
                icarry = c[NVREG - 1]
                below = c - 1
                above = jnp.minimum(c, NS)
                bm1 = jnp.maximum(below - 1, 0)
                cdf_below = jnp.where(
                    below == 0, 0.0, plsc.load_gather(cdf_v, [row, bm1]))
                cdf_above = plsc.load_gather(cdf_v, [row, above - 1])
                eb_below = plsc.load_gather(eb_v, [row, below])
                eb_above = plsc.load_gather(eb_v, [row, above])
                pvec = iota + q * NVREG
                j = jnp.clip(pvec - c, 0, NS)
                uj = plsc.load_gather(u_v, [j])
                denom = jnp.maximum(cdf_above - cdf_below, 1e-37)
                t = jnp.clip((uj - cdf_below) / denom, 0.0, 1.0)
                sample = eb_below + t * (eb_above - eb_below)
                bins = jnp.where(a == 1, eb_below, sample)
                eu = bins * far + (1.0 - bins) * near
                if (q + 1) * NVREG <= NOUT:
                    out_v[r, pl.ds(q * NVREG, NVREG)] = eu
                else:
                    plsc.store_scatter(
                        out_v, [row, jnp.minimum(pvec, NOUT - 1)], eu,
                        mask=pvec < NOUT)
            return carry_unused

        lax.fori_loop(0, CB, ray_body, 0)
        pltpu.sync_copy(out_v, out_hbm.at[pl.ds(base, CB)])
        return _

    lax.fori_loop(0, NCHUNK, chunk_body, 0)


def kernel(weights, existing_bins, nears, fars):
    w2 = weights[..., 0]
    ne = nears[:, 0]
    fa = fars[:, 0]
    u = jnp.linspace(0.0, 1.0 - 1.0 / NB, NB, dtype=jnp.float32)
    u = u + 1.0 / (2 * NB)
    u_pad = jnp.pad(u, (0, 144 - NB))
    return _sampler(w2, existing_bins, ne, fa, u_pad)


# speedup vs baseline: 10.9228x; 1.0132x over previous
import functools

import jax
import jax.numpy as jnp
from jax import lax
from jax.experimental import pallas as pl
from jax.experimental.pallas import tpu as pltpu
from jax.experimental.pallas import tpu_sc as plsc

NUM_RAYS = 65536
NS = 128
NB = NS + 1
NOUT = 2 * NB
NVREG = 16
NOUT_PAD = 272
NW = 32
RAYS_PER_W = NUM_RAYS // NW
CB = 16
NCHUNK = RAYS_PER_W // CB

HIST_PAD = 0.01
EPS = 1e-5

_mesh = plsc.VectorSubcoreMesh(
    core_axis_name="c", subcore_axis_name="s", num_cores=2, num_subcores=16
)


@functools.partial(
    pl.kernel,
    out_type=jax.ShapeDtypeStruct((NUM_RAYS, NOUT), jnp.float32),
    mesh=_mesh,
    compiler_params=pltpu.CompilerParams(needs_layout_passes=False),
    scratch_types=[
        pltpu.VMEM((144,), jnp.float32),
        pltpu.VMEM((CB, NS), jnp.float32),
        pltpu.VMEM((CB, NB), jnp.float32),
        pltpu.VMEM((CB,), jnp.float32),
        pltpu.VMEM((CB,), jnp.float32),
        pltpu.VMEM((CB, NS), jnp.float32),
        pltpu.VMEM((CB, NOUT_PAD), jnp.int32),
        pltpu.VMEM((CB, NOUT), jnp.float32),
    ],
)
def _sampler(w_hbm, eb_hbm, ne_hbm, fa_hbm, u_hbm, out_hbm,
             u_v, w_v, eb_v, ne_v, fa_v, cdf_v, a_v, out_v):
    wid = lax.axis_index("s") * 2 + lax.axis_index("c")
    base0 = wid * RAYS_PER_W

    pltpu.sync_copy(u_hbm, u_v)

    iota = lax.broadcasted_iota(jnp.int32, (NVREG,), 0)
    ones_i = jnp.ones((NVREG,), jnp.int32)
    zeros_i = jnp.zeros((NVREG,), jnp.int32)
    seed0 = jnp.where(iota == 0, 1, 0).astype(jnp.int32)

    def chunk_body(ci, _):
        base = base0 + ci * CB
        pltpu.sync_copy(w_hbm.at[pl.ds(base, CB)], w_v)
        pltpu.sync_copy(eb_hbm.at[pl.ds(base, CB)], eb_v)
        pltpu.sync_copy(ne_hbm.at[pl.ds(base, CB)], ne_v)
        pltpu.sync_copy(fa_hbm.at[pl.ds(base, CB)], fa_v)

        def ray_body(r, carry_unused):
            row = jnp.full((NVREG,), r, jnp.int32)

            a_v[r, pl.ds(0, NVREG)] = seed0
            for q in range(1, NOUT_PAD // NVREG):
                a_v[r, pl.ds(q * NVREG, NVREG)] = zeros_i

            nv = NS // NVREG
            wv = [w_v[r, pl.ds(v * NVREG, NVREG)] + HIST_PAD for v in range(nv)]
            csr = [plsc.cumsum(wv[v]) for v in range(nv)]
            pre = [jnp.float32(0.0)]
            for v in range(nv):
                pre.append(pre[v] + csr[v][NVREG - 1])
            w_sum = pre[nv]
            pad = jnp.maximum(EPS - w_sum, 0.0)
            w_sum2 = w_sum + pad
            padc = pad * (1.0 / NS)
            recip = 1.0 / jnp.full((NVREG,), w_sum2, jnp.float32)
            kvec = (iota + 1).astype(jnp.float32)
            for v in range(nv):
                cs = (csr[v] + pre[v] + (kvec + (v * NVREG)) * padc) * recip
                cdfv = jnp.minimum(cs, 1.0)
                cdf_v[r, pl.ds(v * NVREG, NVREG)] = cdfv
                y = cdfv * jnp.float32(NB) - 0.5
                tr = y.astype(jnp.int32)
                m = tr + jnp.where(tr.astype(jnp.float32) < y, 1, 0)
                pos = iota + (v * NVREG + 1) + m
                plsc.store_scatter(a_v, [row, pos], ones_i)

            near = plsc.load_gather(ne_v, [row])
            far = plsc.load_gather(fa_v, [row])
            nq = NOUT_PAD // NVREG
            avs = [a_v[r, pl.ds(q * NVREG, NVREG)] for q in range(nq)]
            csq = [plsc.cumsum(avs[q]) for q in range(nq)]
            ipre = [jnp.int32(0)]
            for q in range(nq):
                ipre.append(ipre[q] + csq[q][NVREG - 1])
            for q in range(nq):
                a = avs[q]
                c = csq[q] + ipre[q]
                below = c - 1
                above = jnp.minimum(c, NS)
                bm1 = jnp.maximum(below - 1, 0)
                cdf_below = jnp.where(
                    below == 0, 0.0, plsc.load_gather(cdf_v, [row, bm1]))
                cdf_above = plsc.load_gather(cdf_v, [row, above - 1])
                eb_below = plsc.load_gather(eb_v, [row, below])
                eb_above = plsc.load_gather(eb_v, [row, above])
                pvec = iota + q * NVREG
                j = jnp.clip(pvec - c, 0, NS)
                uj = plsc.load_gather(u_v, [j])
                denom = jnp.maximum(cdf_above - cdf_below, 1e-37)
                t = jnp.clip((uj - cdf_below) / denom, 0.0, 1.0)
                sample = eb_below + t * (eb_above - eb_below)
                bins = jnp.where(a == 1, eb_below, sample)
                eu = bins * far + (1.0 - bins) * near
                if (q + 1) * NVREG <= NOUT:
                    out_v[r, pl.ds(q * NVREG, NVREG)] = eu
                else:
                    plsc.store_scatter(
                        out_v, [row, jnp.minimum(pvec, NOUT - 1)], eu,
                        mask=pvec < NOUT)
            return carry_unused

        lax.fori_loop(0, CB, ray_body, 0)
        pltpu.sync_copy(out_v, out_hbm.at[pl.ds(base, CB)])
        return _

    lax.fori_loop(0, NCHUNK, chunk_body, 0)


def kernel(weights, existing_bins, nears, fars):
    w2 = weights[..., 0]
    ne = nears[:, 0]
    fa = fars[:, 0]
    u = jnp.linspace(0.0, 1.0 - 1.0 / NB, NB, dtype=jnp.float32)
    u = u + 1.0 / (2 * NB)
    u_pad = jnp.pad(u, (0, 144 - NB))
    return _sampler(w2, existing_bins, ne, fa, u_pad)


# speedup vs baseline: 11.1010x; 1.0163x over previous
import functools

import jax
import jax.numpy as jnp
from jax import lax
from jax.experimental import pallas as pl
from jax.experimental.pallas import tpu as pltpu
from jax.experimental.pallas import tpu_sc as plsc

NUM_RAYS = 65536
NS = 128
NB = NS + 1
NOUT = 2 * NB
NVREG = 16
NOUT_PAD = 272
NW = 32
RAYS_PER_W = NUM_RAYS // NW
CB = 16
NCHUNK = RAYS_PER_W // CB

HIST_PAD = 0.01
EPS = 1e-5

_mesh = plsc.VectorSubcoreMesh(
    core_axis_name="c", subcore_axis_name="s", num_cores=2, num_subcores=16
)


@functools.partial(
    pl.kernel,
    out_type=jax.ShapeDtypeStruct((NUM_RAYS, NOUT), jnp.float32),
    mesh=_mesh,
    compiler_params=pltpu.CompilerParams(needs_layout_passes=False),
    scratch_types=[
        pltpu.VMEM((144,), jnp.float32),
        pltpu.VMEM((CB, NS), jnp.float32),
        pltpu.VMEM((CB, NB), jnp.float32),
        pltpu.VMEM((CB,), jnp.float32),
        pltpu.VMEM((CB,), jnp.float32),
        pltpu.VMEM((CB, NS), jnp.float32),
        pltpu.VMEM((CB, NOUT_PAD), jnp.int32),
        pltpu.VMEM((CB, NOUT), jnp.float32),
    ],
)
def _sampler(w_hbm, eb_hbm, ne_hbm, fa_hbm, u_hbm, out_hbm,
             u_v, w_v, eb_v, ne_v, fa_v, cdf_v, a_v, out_v):
    wid = lax.axis_index("s") * 2 + lax.axis_index("c")
    base0 = wid * RAYS_PER_W

    pltpu.sync_copy(u_hbm, u_v)

    iota = lax.broadcasted_iota(jnp.int32, (NVREG,), 0)
    ones_i = jnp.ones((NVREG,), jnp.int32)
    zeros_i = jnp.zeros((NVREG,), jnp.int32)
    seed0 = jnp.where(iota == 0, 1, 0).astype(jnp.int32)

    def chunk_body(ci, _):
        base = base0 + ci * CB
        pltpu.sync_copy(w_hbm.at[pl.ds(base, CB)], w_v)
        pltpu.sync_copy(eb_hbm.at[pl.ds(base, CB)], eb_v)
        pltpu.sync_copy(ne_hbm.at[pl.ds(base, CB)], ne_v)
        pltpu.sync_copy(fa_hbm.at[pl.ds(base, CB)], fa_v)

        @plsc.parallel_loop(0, CB, 1, unroll=2)
        def ray_body(r):
            row = jnp.full((NVREG,), r, jnp.int32)

            a_v[r, pl.ds(0, NVREG)] = seed0
            for q in range(1, NOUT_PAD // NVREG):
                a_v[r, pl.ds(q * NVREG, NVREG)] = zeros_i

            nv = NS // NVREG
            wv = [w_v[r, pl.ds(v * NVREG, NVREG)] + HIST_PAD for v in range(nv)]
            csr = [plsc.cumsum(wv[v]) for v in range(nv)]
            pre = [jnp.float32(0.0)]
            for v in range(nv):
                pre.append(pre[v] + csr[v][NVREG - 1])
            w_sum = pre[nv]
            pad = jnp.maximum(EPS - w_sum, 0.0)
            w_sum2 = w_sum + pad
            padc = pad * (1.0 / NS)
            recip = 1.0 / jnp.full((NVREG,), w_sum2, jnp.float32)
            kvec = (iota + 1).astype(jnp.float32)
            for v in range(nv):
                cs = (csr[v] + pre[v] + (kvec + (v * NVREG)) * padc) * recip
                cdfv = jnp.minimum(cs, 1.0)
                cdf_v[r, pl.ds(v * NVREG, NVREG)] = cdfv
                y = cdfv * jnp.float32(NB) - 0.5
                tr = y.astype(jnp.int32)
                m = tr + jnp.where(tr.astype(jnp.float32) < y, 1, 0)
                pos = iota + (v * NVREG + 1) + m
                plsc.store_scatter(a_v, [row, pos], ones_i)

            near = plsc.load_gather(ne_v, [row])
            far = plsc.load_gather(fa_v, [row])
            nq = NOUT_PAD // NVREG
            avs = [a_v[r, pl.ds(q * NVREG, NVREG)] for q in range(nq)]
            csq = [plsc.cumsum(avs[q]) for q in range(nq)]
            ipre = [jnp.int32(0)]
            for q in range(nq):
                ipre.append(ipre[q] + csq[q][NVREG - 1])
            for q in range(nq):
                a = avs[q]
                c = csq[q] + ipre[q]
                below = c - 1
                above = jnp.minimum(c, NS)
                bm1 = jnp.maximum(below - 1, 0)
                cdf_below = jnp.where(
                    below == 0, 0.0, plsc.load_gather(cdf_v, [row, bm1]))
                cdf_above = plsc.load_gather(cdf_v, [row, above - 1])
                eb_below = plsc.load_gather(eb_v, [row, below])
                eb_above = plsc.load_gather(eb_v, [row, above])
                pvec = iota + q * NVREG
                j = jnp.clip(pvec - c, 0, NS)
                uj = plsc.load_gather(u_v, [j])
                denom = jnp.maximum(cdf_above - cdf_below, 1e-37)
                t = jnp.clip((uj - cdf_below) / denom, 0.0, 1.0)
                sample = eb_below + t * (eb_above - eb_below)
                bins = jnp.where(a == 1, eb_below, sample)
                eu = bins * far + (1.0 - bins) * near
                if (q + 1) * NVREG <= NOUT:
                    out_v[r, pl.ds(q * NVREG, NVREG)] = eu
                else:
                    plsc.store_scatter(
                        out_v, [row, jnp.minimum(pvec, NOUT - 1)], eu,
                        mask=pvec < NOUT)

        pltpu.sync_copy(out_v, out_hbm.at[pl.ds(base, CB)])
        return _

    lax.fori_loop(0, NCHUNK, chunk_body, 0)


def kernel(weights, existing_bins, nears, fars):
    w2 = weights[..., 0]
    ne = nears[:, 0]
    fa = fars[:, 0]
    u = jnp.linspace(0.0, 1.0 - 1.0 / NB, NB, dtype=jnp.float32)
    u = u + 1.0 / (2 * NB)
    u_pad = jnp.pad(u, (0, 144 - NB))
    return _sampler(w2, existing_bins, ne, fa, u_pad)


# speedup vs baseline: 11.5979x; 1.0448x over previous
import functools

import jax
import jax.numpy as jnp
from jax import lax
from jax.experimental import pallas as pl
from jax.experimental.pallas import tpu as pltpu
from jax.experimental.pallas import tpu_sc as plsc

NUM_RAYS = 65536
NS = 128
NB = NS + 1
NOUT = 2 * NB
NVREG = 16
NOUT_PAD = 272
NW = 32
RAYS_PER_W = NUM_RAYS // NW
CB = 16
NCHUNK = RAYS_PER_W // CB

HIST_PAD = 0.01
EPS = 1e-5

_mesh = plsc.VectorSubcoreMesh(
    core_axis_name="c", subcore_axis_name="s", num_cores=2, num_subcores=16
)


@functools.partial(
    pl.kernel,
    out_type=jax.ShapeDtypeStruct((NUM_RAYS, NOUT), jnp.float32),
    mesh=_mesh,
    compiler_params=pltpu.CompilerParams(needs_layout_passes=False),
    scratch_types=[
        pltpu.VMEM((CB, NS), jnp.float32),
        pltpu.VMEM((CB, NB), jnp.float32),
        pltpu.VMEM((CB,), jnp.float32),
        pltpu.VMEM((CB,), jnp.float32),
        pltpu.VMEM((CB, 144), jnp.float32),
        pltpu.VMEM((CB, NOUT_PAD), jnp.int32),
        pltpu.VMEM((CB, NOUT), jnp.float32),
    ],
)
def _sampler(w_hbm, eb_hbm, ne_hbm, fa_hbm, out_hbm,
             w_v, eb_v, ne_v, fa_v, cdf_v, a_v, out_v):
    wid = lax.axis_index("s") * 2 + lax.axis_index("c")
    base0 = wid * RAYS_PER_W

    iota = lax.broadcasted_iota(jnp.int32, (NVREG,), 0)
    ones_i = jnp.ones((NVREG,), jnp.int32)
    zeros_i = jnp.zeros((NVREG,), jnp.int32)
    seed0 = jnp.where(iota == 0, 1, 0).astype(jnp.int32)

    zeros_f = jnp.zeros((NVREG,), jnp.float32)

    def chunk_body(ci, _):
        base = base0 + ci * CB
        pltpu.sync_copy(w_hbm.at[pl.ds(base, CB)], w_v)
        pltpu.sync_copy(eb_hbm.at[pl.ds(base, CB)], eb_v)
        pltpu.sync_copy(ne_hbm.at[pl.ds(base, CB)], ne_v)
        pltpu.sync_copy(fa_hbm.at[pl.ds(base, CB)], fa_v)
        plsc.store_scatter(cdf_v, [iota, zeros_i], zeros_f)

        @plsc.parallel_loop(0, CB, 1, unroll=2)
        def ray_body(r):
            row = jnp.full((NVREG,), r, jnp.int32)

            a_v[r, pl.ds(0, NVREG)] = seed0
            for q in range(1, NOUT_PAD // NVREG):
                a_v[r, pl.ds(q * NVREG, NVREG)] = zeros_i

            nv = NS // NVREG
            wv = [w_v[r, pl.ds(v * NVREG, NVREG)] + HIST_PAD for v in range(nv)]
            csr = [plsc.cumsum(wv[v]) for v in range(nv)]
            pre = [jnp.float32(0.0)]
            for v in range(nv):
                pre.append(pre[v] + csr[v][NVREG - 1])
            w_sum = pre[nv]
            pad = jnp.maximum(EPS - w_sum, 0.0)
            w_sum2 = w_sum + pad
            padc = pad * (1.0 / NS)
            recip = 1.0 / jnp.full((NVREG,), w_sum2, jnp.float32)
            kvec = (iota + 1).astype(jnp.float32)
            for v in range(nv):
                cs = (csr[v] + pre[v] + (kvec + (v * NVREG)) * padc) * recip
                cdfv = jnp.minimum(cs, 1.0)
                cdf_v[r, pl.ds(1 + v * NVREG, NVREG)] = cdfv
                m = (cdfv * jnp.float32(NB) + 0.5).astype(jnp.int32)
                pos = iota + (v * NVREG + 1) + m
                plsc.store_scatter(a_v, [row, pos], ones_i)

            near = plsc.load_gather(ne_v, [row])
            fmn = plsc.load_gather(fa_v, [row]) - near
            nq = NOUT_PAD // NVREG
            avs = [a_v[r, pl.ds(q * NVREG, NVREG)] for q in range(nq)]
            csq = [plsc.cumsum(avs[q]) for q in range(nq)]
            ipre = [jnp.int32(0)]
            for q in range(nq):
                ipre.append(ipre[q] + csq[q][NVREG - 1])
            for q in range(nq):
                a = avs[q]
                c = csq[q] + ipre[q]
                below = c - 1
                above = jnp.minimum(c, NS)
                cdf_below = plsc.load_gather(cdf_v, [row, below])
                cdf_above = plsc.load_gather(cdf_v, [row, above])
                eb_below = plsc.load_gather(eb_v, [row, below])
                eb_above = plsc.load_gather(eb_v, [row, above])
                pvec = iota + q * NVREG
                uj = ((pvec - c).astype(jnp.float32) * jnp.float32(1.0 / NB)
                      + jnp.float32(1.0 / (2 * NB)))
                denom = jnp.maximum(cdf_above - cdf_below, 1e-37)
                t = jnp.clip((uj - cdf_below) / denom, 0.0, 1.0)
                sample = eb_below + t * (eb_above - eb_below)
                bins = jnp.where(a == 1, eb_below, sample)
                eu = near + bins * fmn
                if (q + 1) * NVREG <= NOUT:
                    out_v[r, pl.ds(q * NVREG, NVREG)] = eu
                else:
                    plsc.store_scatter(
                        out_v, [row, jnp.minimum(pvec, NOUT - 1)], eu,
                        mask=pvec < NOUT)

        pltpu.sync_copy(out_v, out_hbm.at[pl.ds(base, CB)])
        return _

    lax.fori_loop(0, NCHUNK, chunk_body, 0)


def kernel(weights, existing_bins, nears, fars):
    w2 = weights[..., 0]
    ne = nears[:, 0]
    fa = fars[:, 0]
    return _sampler(w2, existing_bins, ne, fa)
